# baseline (speedup 1.0000x reference)
import functools

import jax
import jax.numpy as jnp
from jax import lax
from jax.experimental import pallas as pl
from jax.experimental.pallas import tpu as pltpu

F32 = jnp.float32


def _fc1_kernel(x_ref, w1t_ref, b1_ref, h_ref, stats_ref):
    xa = x_ref[0:8, :]
    xb = x_ref[8:10, :]
    hs = []
    for f in range(6):
        wa = w1t_ref[0:8, f:f + 1]
        wb = w1t_ref[8:10, f:f + 1]
        hf = (jnp.sum(xa * wa, axis=0, keepdims=True) +
              jnp.sum(xb * wb, axis=0, keepdims=True))
        hs.append(hf)
    h = jnp.concatenate(hs, axis=0)
    h = jnp.maximum(h + b1_ref[...], 0.0)
    h_ref[...] = h.astype(jnp.bfloat16)
    s = jnp.sum(h, axis=1, keepdims=True)
    q = jnp.sum(h * h, axis=1, keepdims=True)
    lane = lax.broadcasted_iota(jnp.int32, (6, 128), 1)
    stats_ref[0] = jnp.where(lane == 0, s, 0.0) + jnp.where(lane == 1, q, 0.0)


def _fc2_kernel(h_ref, stats_ref, g_ref, bt_ref, w2_ref, b2_ref, o_ref, *,
                batch, eps):
    tot = jnp.sum(stats_ref[...], axis=0)
    s = tot[:, 0:1]
    q = tot[:, 1:2]
    mean = s * (1.0 / batch)
    var = jnp.maximum(q * (1.0 / batch) - mean * mean, 0.0)
    scale = g_ref[...] * jax.lax.rsqrt(var + eps)
    shift = bt_ref[...] - mean * scale
    w2e = w2_ref[...] * scale
    b2e = jnp.sum(w2_ref[...] * shift) + b2_ref[0, 0]
    y = (jnp.sum(h_ref[...].astype(F32) * w2e, axis=0, keepdims=True)
         + b2e)
    o_ref[...] = 0.5 * jnp.tanh(0.5 * y) + 0.5


def _pick_tile(b, cap):
    tb = cap
    while tb > 1 and b % tb:
        tb //= 2
    return tb


def kernel(x, w1, b1, gamma, beta, w2, b2):
    B = x.shape[0]
    eps = 1e-5
    xt = x.astype(F32).T
    w1t = w1.astype(F32).T
    b1c = b1.astype(F32).reshape(6, 1)

    tb = _pick_tile(B, 131072)
    nb = B // tb
    parallel = pltpu.CompilerParams(dimension_semantics=("parallel",))

    h, stats = pl.pallas_call(
        _fc1_kernel,
        out_shape=[jax.ShapeDtypeStruct((6, B), jnp.bfloat16),
                   jax.ShapeDtypeStruct((nb, 6, 128), F32)],
        grid=(nb,),
        in_specs=[pl.BlockSpec((10, tb), lambda i: (0, i)),
                  pl.BlockSpec((10, 6), lambda i: (0, 0)),
                  pl.BlockSpec((6, 1), lambda i: (0, 0))],
        out_specs=[pl.BlockSpec((6, tb), lambda i: (0, i)),
                   pl.BlockSpec((1, 6, 128), lambda i: (i, 0, 0))],
        compiler_params=parallel,
    )(xt, w1t, b1c)

    tb2 = _pick_tile(B, 131072)
    nb2 = B // tb2
    gc = gamma.astype(F32).reshape(6, 1)
    btc = beta.astype(F32).reshape(6, 1)
    w2c = w2.astype(F32).reshape(6, 1)
    b2c = b2.astype(F32).reshape(1, 1)

    out = pl.pallas_call(
        functools.partial(_fc2_kernel, batch=float(B), eps=eps),
        out_shape=jax.ShapeDtypeStruct((1, B), F32),
        grid=(nb2,),
        in_specs=[pl.BlockSpec((6, tb2), lambda i: (0, i)),
                  pl.BlockSpec((nb, 6, 128), lambda i: (0, 0, 0)),
                  pl.BlockSpec((6, 1), lambda i: (0, 0)),
                  pl.BlockSpec((6, 1), lambda i: (0, 0)),
                  pl.BlockSpec((6, 1), lambda i: (0, 0)),
                  pl.BlockSpec((1, 1), lambda i: (0, 0))],
        out_specs=pl.BlockSpec((1, tb2), lambda i: (0, i)),
        compiler_params=parallel,
    )(h, stats, gc, btc, w2c, b2c)

    return out.T


# speedup vs baseline: 1.0108x; 1.0108x over previous
import functools

import jax
import jax.numpy as jnp
from jax import lax
from jax.experimental import pallas as pl
from jax.experimental.pallas import tpu as pltpu

F32 = jnp.float32


def _fc1_kernel(x_ref, w1t_ref, b1_ref, h_ref, stats_ref):
    xa = x_ref[0:8, :]
    xb = x_ref[8:10, :]
    hs = []
    for f in range(6):
        wa = w1t_ref[0:8, f:f + 1]
        wb = w1t_ref[8:10, f:f + 1]
        hf = (jnp.sum(xa * wa, axis=0, keepdims=True) +
              jnp.sum(xb * wb, axis=0, keepdims=True))
        hs.append(hf)
    h = jnp.concatenate(hs, axis=0)
    h = jnp.maximum(h + b1_ref[...], 0.0)
    h_ref[...] = h.astype(jnp.bfloat16)
    s = jnp.sum(h, axis=1, keepdims=True)
    q = jnp.sum(h * h, axis=1, keepdims=True)
    lane = lax.broadcasted_iota(jnp.int32, (6, 128), 1)
    stats_ref[0] = jnp.where(lane == 0, s, 0.0) + jnp.where(lane == 1, q, 0.0)


def _fc2_kernel(h_ref, stats_ref, g_ref, bt_ref, w2_ref, b2_ref, o_ref, *,
                batch, eps):
    tot = jnp.sum(stats_ref[...], axis=0)
    s = tot[:, 0:1]
    q = tot[:, 1:2]
    mean = s * (1.0 / batch)
    var = jnp.maximum(q * (1.0 / batch) - mean * mean, 0.0)
    scale = g_ref[...] * jax.lax.rsqrt(var + eps)
    shift = bt_ref[...] - mean * scale
    w2e = w2_ref[...] * scale
    b2e = jnp.sum(w2_ref[...] * shift) + b2_ref[0, 0]
    y = (jnp.sum(h_ref[...].astype(F32) * w2e, axis=0, keepdims=True)
         + b2e)
    o_ref[...] = 0.5 * jnp.tanh(0.5 * y) + 0.5


def _pick_tile(b, cap):
    tb = cap
    while tb > 1 and b % tb:
        tb //= 2
    return tb


def kernel(x, w1, b1, gamma, beta, w2, b2):
    B = x.shape[0]
    eps = 1e-5
    xt = x.astype(F32).T
    w1t = w1.astype(F32).T
    b1c = b1.astype(F32).reshape(6, 1)

    tb = _pick_tile(B, 65536)
    nb = B // tb
    parallel = pltpu.CompilerParams(dimension_semantics=("parallel",))

    h, stats = pl.pallas_call(
        _fc1_kernel,
        out_shape=[jax.ShapeDtypeStruct((6, B), jnp.bfloat16),
                   jax.ShapeDtypeStruct((nb, 6, 128), F32)],
        grid=(nb,),
        in_specs=[pl.BlockSpec((10, tb), lambda i: (0, i)),
                  pl.BlockSpec((10, 6), lambda i: (0, 0)),
                  pl.BlockSpec((6, 1), lambda i: (0, 0))],
        out_specs=[pl.BlockSpec((6, tb), lambda i: (0, i)),
                   pl.BlockSpec((1, 6, 128), lambda i: (i, 0, 0))],
        compiler_params=parallel,
    )(xt, w1t, b1c)

    tb2 = _pick_tile(B, 131072)
    nb2 = B // tb2
    gc = gamma.astype(F32).reshape(6, 1)
    btc = beta.astype(F32).reshape(6, 1)
    w2c = w2.astype(F32).reshape(6, 1)
    b2c = b2.astype(F32).reshape(1, 1)

    out = pl.pallas_call(
        functools.partial(_fc2_kernel, batch=float(B), eps=eps),
        out_shape=jax.ShapeDtypeStruct((1, B), F32),
        grid=(nb2,),
        in_specs=[pl.BlockSpec((6, tb2), lambda i: (0, i)),
                  pl.BlockSpec((nb, 6, 128), lambda i: (0, 0, 0)),
                  pl.BlockSpec((6, 1), lambda i: (0, 0)),
                  pl.BlockSpec((6, 1), lambda i: (0, 0)),
                  pl.BlockSpec((6, 1), lambda i: (0, 0)),
                  pl.BlockSpec((1, 1), lambda i: (0, 0))],
        out_specs=pl.BlockSpec((1, tb2), lambda i: (0, i)),
        compiler_params=parallel,
    )(h, stats, gc, btc, w2c, b2c)

    return out.T
